# baseline (device time: 15334 ns/iter reference)
import functools

import jax
import jax.numpy as jnp
from jax import lax
from jax.experimental import pallas as pl
from jax.experimental.pallas import tpu as pltpu


def kernel(x):
    m, n = x.shape

    def body(x_ref, out_ref, row_recv, col_recv, row_sems, col_sems):
        my_x = lax.axis_index("x")
        my_y = lax.axis_index("y")
        nbr_x = (1 - my_x, my_y)
        nbr_y = (my_x, 1 - my_y)

        barrier_sem = pltpu.get_barrier_semaphore()
        pl.semaphore_signal(barrier_sem, inc=1, device_id=nbr_x,
                            device_id_type=pl.DeviceIdType.MESH)
        pl.semaphore_signal(barrier_sem, inc=1, device_id=nbr_y,
                            device_id_type=pl.DeviceIdType.MESH)
        pl.semaphore_wait(barrier_sem, 2)

        src_row = pl.multiple_of(jnp.where(my_x == 0, m - 8, 0), 8)
        src_col = pl.multiple_of(jnp.where(my_y == 0, n - 128, 0), 128)

        row_rdma = pltpu.make_async_remote_copy(
            src_ref=x_ref.at[pl.ds(src_row, 8), :],
            dst_ref=row_recv,
            send_sem=row_sems.at[0],
            recv_sem=row_sems.at[1],
            device_id=nbr_x,
            device_id_type=pl.DeviceIdType.MESH,
        )
        col_rdma = pltpu.make_async_remote_copy(
            src_ref=x_ref.at[:, pl.ds(src_col, 128)],
            dst_ref=col_recv,
            send_sem=col_sems.at[0],
            recv_sem=col_sems.at[1],
            device_id=nbr_y,
            device_id_type=pl.DeviceIdType.MESH,
        )
        row_rdma.start()
        col_rdma.start()

        xv = x_ref[:, :]
        zrow = jnp.zeros((1, n), xv.dtype)
        zcol = jnp.zeros((m, 1), xv.dtype)
        nv = jnp.concatenate([zrow, xv[:-1, :]], axis=0)
        sv = jnp.concatenate([xv[1:, :], zrow], axis=0)
        wv = jnp.concatenate([zcol, xv[:, :-1]], axis=1)
        ev = jnp.concatenate([xv[:, 1:], zcol], axis=1)
        out_ref[:, :] = 0.5 * xv + 0.125 * (nv + sv + wv + ev)

        row_rdma.wait()
        rrow = jnp.where(my_x == 1, row_recv[7:8, :], row_recv[0:1, :])
        rb = pl.multiple_of(jnp.where(my_x == 1, 0, m - 8), 8)
        tgt_sub = jnp.where(my_x == 1, 0, 7)
        sub_i8 = lax.broadcasted_iota(jnp.int32, (8, n), 0)
        add = jnp.where(sub_i8 == tgt_sub,
                        0.125 * jnp.broadcast_to(rrow, (8, n)), 0.0)
        out_ref[pl.ds(rb, 8), :] = out_ref[pl.ds(rb, 8), :] + add

        col_rdma.wait()
        rcol = jnp.where(my_y == 1, col_recv[:, 127:128], col_recv[:, 0:1])
        cb = pl.multiple_of(jnp.where(my_y == 1, 0, n - 128), 128)
        tgt_lane = jnp.where(my_y == 1, 0, 127)
        lane_c = lax.broadcasted_iota(jnp.int32, (m, 128), 1)
        cadd = jnp.where(lane_c == tgt_lane,
                         0.125 * jnp.broadcast_to(rcol, (m, 128)), 0.0)
        out_ref[:, pl.ds(cb, 128)] = out_ref[:, pl.ds(cb, 128)] + cadd

        gb_r = pl.multiple_of(jnp.where(my_x == 0, 0, m - 8), 8)
        gb_sub = jnp.where(my_x == 0, 0, 7)
        xrow = x_ref[pl.ds(gb_r, 8), :]
        out_ref[pl.ds(gb_r, 8), :] = jnp.where(
            sub_i8 == gb_sub, xrow, out_ref[pl.ds(gb_r, 8), :])

        gb_c = pl.multiple_of(jnp.where(my_y == 0, 0, n - 128), 128)
        gb_lane = jnp.where(my_y == 0, 0, 127)
        xcol = x_ref[:, pl.ds(gb_c, 128)]
        out_ref[:, pl.ds(gb_c, 128)] = jnp.where(
            lane_c == gb_lane, xcol, out_ref[:, pl.ds(gb_c, 128)])

        @functools.partial(pl.run_scoped, sem2=pltpu.SemaphoreType.REGULAR)
        def _(sem2):
            pl.semaphore_signal(sem2, inc=1, device_id=nbr_x,
                                device_id_type=pl.DeviceIdType.MESH)
            pl.semaphore_signal(sem2, inc=1, device_id=nbr_y,
                                device_id_type=pl.DeviceIdType.MESH)
            pl.semaphore_wait(sem2, 2)

    return pl.pallas_call(
        body,
        out_shape=jax.ShapeDtypeStruct((m, n), x.dtype),
        in_specs=[pl.BlockSpec(memory_space=pltpu.VMEM)],
        out_specs=pl.BlockSpec(memory_space=pltpu.VMEM),
        scratch_shapes=[
            pltpu.VMEM((8, n), x.dtype),
            pltpu.VMEM((m, 128), x.dtype),
            pltpu.SemaphoreType.DMA((2,)),
            pltpu.SemaphoreType.DMA((2,)),
        ],
        compiler_params=pltpu.CompilerParams(collective_id=0),
    )(x)
